# baseline (speedup 1.0000x reference)
import functools

import jax
import jax.numpy as jnp
from jax import lax
from jax.experimental import pallas as pl
from jax.experimental.pallas import tpu as pltpu
from jax.experimental.pallas import tpu_sc as plsc

N_CLUSTERS = 64
USER_EMBED_DIM = 64
BATCH = 16384

NC = 2
NS = 16
NW = NC * NS
B_PER_W = BATCH // NW
IDX_CHUNK = 128
N_CHUNK = B_PER_W // IDX_CHUNK


def _sc_gather_body(idx_hbm, table_hbm, out_hbm, idx_v, rows_v, sem):
  wid = lax.axis_index("s") * NC + lax.axis_index("c")
  base = wid * B_PER_W
  pltpu.sync_copy(idx_hbm.at[pl.ds(base, B_PER_W)], idx_v)
  copies = []
  for j in range(N_CHUNK):
    copies.append(
        pltpu.async_copy(
            table_hbm.at[idx_v.at[pl.ds(j * IDX_CHUNK, IDX_CHUNK)]],
            rows_v.at[pl.ds(j * IDX_CHUNK, IDX_CHUNK)],
            sem,
        )
    )
  for c in copies:
    c.wait()
  pltpu.sync_copy(rows_v, out_hbm.at[pl.ds(base, B_PER_W)])


_sc_gather = pl.kernel(
    _sc_gather_body,
    out_type=jax.ShapeDtypeStruct((BATCH, USER_EMBED_DIM), jnp.float32),
    mesh=plsc.VectorSubcoreMesh(core_axis_name="c", subcore_axis_name="s"),
    scratch_types=[
        pltpu.VMEM((B_PER_W,), jnp.int32),
        pltpu.VMEM((B_PER_W, USER_EMBED_DIM), jnp.float32),
        pltpu.SemaphoreType.DMA,
    ],
    compiler_params=pltpu.CompilerParams(use_tc_tiling_on_sc=False),
)


def _tc_body(rows_ref, ce_ref, out_ref):
  w = rows_ref[...]
  w = w - jnp.max(w, axis=1, keepdims=True)
  e = jnp.exp(w)
  p = e / jnp.sum(e, axis=1, keepdims=True)
  out_ref[...] = jnp.dot(p, ce_ref[...], preferred_element_type=jnp.float32)


TC_BLOCK = 2048


def _tc_softmax_matmul(rows, ce):
  grid = BATCH // TC_BLOCK
  return pl.pallas_call(
      _tc_body,
      grid=(grid,),
      in_specs=[
          pl.BlockSpec((TC_BLOCK, N_CLUSTERS), lambda i: (i, 0)),
          pl.BlockSpec((N_CLUSTERS, USER_EMBED_DIM), lambda i: (0, 0)),
      ],
      out_specs=pl.BlockSpec((TC_BLOCK, USER_EMBED_DIM), lambda i: (i, 0)),
      out_shape=jax.ShapeDtypeStruct((BATCH, USER_EMBED_DIM), jnp.float32),
  )(rows, ce)


@jax.jit
def kernel(user_identifiers, user_weight, cluster_embedding):
  idx = user_identifiers.astype(jnp.int32)
  rows = _sc_gather(idx, user_weight)
  out = _tc_softmax_matmul(rows, cluster_embedding)
  return out.reshape(BATCH, 1, USER_EMBED_DIM)


# speedup vs baseline: 1.6768x; 1.6768x over previous
import functools

import jax
import jax.numpy as jnp
from jax import lax
from jax.experimental import pallas as pl
from jax.experimental.pallas import tpu as pltpu
from jax.experimental.pallas import tpu_sc as plsc

N_CLUSTERS = 64
USER_EMBED_DIM = 64
BATCH = 16384

NC = 2
NS = 16
NW = NC * NS
B_PER_W = BATCH // NW
FIRE = 16
N_GROUP = B_PER_W // FIRE


def _sc_gather_body(idx_hbm, table_hbm, out_hbm, idx_v, rows_v, sem):
  wid = lax.axis_index("s") * NC + lax.axis_index("c")
  base = wid * B_PER_W

  pltpu.sync_copy(idx_hbm.at[pl.ds(base, B_PER_W)], idx_v)

  def fire_group(g):
    vec = idx_v[pl.ds(g * FIRE, FIRE)]
    for r in range(FIRE):
      rid = vec[r]
      pltpu.async_copy(
          table_hbm.at[pl.ds(rid, 1), :],
          rows_v.at[pl.ds(g * FIRE + r, 1), :],
          sem,
      )

  def drain_group(g):
    pltpu.make_async_copy(
        table_hbm.at[pl.ds(0, FIRE), :],
        rows_v.at[pl.ds(g * FIRE, FIRE), :],
        sem,
    ).wait()

  def body(g, _):
    fire_group(g)

    @pl.when(g > 0)
    def _():
      drain_group(g - 1)

    return 0

  lax.fori_loop(0, N_GROUP, body, 0)
  drain_group(N_GROUP - 1)

  pltpu.sync_copy(rows_v, out_hbm.at[pl.ds(base, B_PER_W)])


_sc_gather = pl.kernel(
    _sc_gather_body,
    out_type=jax.ShapeDtypeStruct((BATCH, USER_EMBED_DIM), jnp.float32),
    mesh=plsc.VectorSubcoreMesh(core_axis_name="c", subcore_axis_name="s"),
    scratch_types=[
        pltpu.VMEM((B_PER_W,), jnp.int32),
        pltpu.VMEM((B_PER_W, USER_EMBED_DIM), jnp.float32),
        pltpu.SemaphoreType.DMA,
    ],
)


def _tc_body(rows_ref, ce_ref, out_ref):
  w = rows_ref[...]
  w = w - jnp.max(w, axis=1, keepdims=True)
  e = jnp.exp(w)
  p = e / jnp.sum(e, axis=1, keepdims=True)
  out_ref[...] = jnp.dot(p, ce_ref[...], preferred_element_type=jnp.float32)


TC_BLOCK = 2048


def _tc_softmax_matmul(rows, ce):
  grid = BATCH // TC_BLOCK
  return pl.pallas_call(
      _tc_body,
      grid=(grid,),
      in_specs=[
          pl.BlockSpec((TC_BLOCK, N_CLUSTERS), lambda i: (i, 0)),
          pl.BlockSpec((N_CLUSTERS, USER_EMBED_DIM), lambda i: (0, 0)),
      ],
      out_specs=pl.BlockSpec((TC_BLOCK, USER_EMBED_DIM), lambda i: (i, 0)),
      out_shape=jax.ShapeDtypeStruct((BATCH, USER_EMBED_DIM), jnp.float32),
  )(rows, ce)


@jax.jit
def kernel(user_identifiers, user_weight, cluster_embedding):
  idx = user_identifiers.astype(jnp.int32)
  rows = _sc_gather(idx, user_weight)
  out = _tc_softmax_matmul(rows, cluster_embedding)
  return out.reshape(BATCH, 1, USER_EMBED_DIM)
